# baseline (speedup 1.0000x reference)
import jax
import jax.numpy as jnp
from jax.experimental import pallas as pl
from jax.experimental.pallas import tpu as pltpu

_TOKENS = 16384
_DIM = 2048
_EXPERTS = 64
_BLOCK_T = 512


def _router_body(x_ref, w_ref, b_ref, out_ref):
    out_ref[...] = jax.lax.dot_general(
        w_ref[...],
        x_ref[...],
        dimension_numbers=(((1,), (1,)), ((), ())),
        preferred_element_type=jnp.float32,
    ) + b_ref[...]


@jax.jit
def kernel(x, W, b):
    grid = (_TOKENS // _BLOCK_T,)
    out_t = pl.pallas_call(
        _router_body,
        grid=grid,
        in_specs=[
            pl.BlockSpec((_BLOCK_T, _DIM), lambda i: (i, 0)),
            pl.BlockSpec((_EXPERTS, _DIM), lambda i: (0, 0)),
            pl.BlockSpec((_EXPERTS, 1), lambda i: (0, 0)),
        ],
        out_specs=pl.BlockSpec((_EXPERTS, _BLOCK_T), lambda i: (0, i)),
        out_shape=jax.ShapeDtypeStruct((_EXPERTS, _TOKENS), jnp.float32),
        compiler_params=pltpu.CompilerParams(
            dimension_semantics=("arbitrary",),
        ),
    )(x, W, b.reshape(_EXPERTS, 1))
    return out_t.T


# speedup vs baseline: 1.1430x; 1.1430x over previous
import jax
import jax.numpy as jnp
from jax.experimental import pallas as pl
from jax.experimental.pallas import tpu as pltpu

_TOKENS = 16384
_DIM = 2048
_EXPERTS = 64
_BLOCK_T = 1024


def _router_body(x_ref, w_ref, b_ref, out_ref):
    i = pl.program_id(0)
    out_ref[:, pl.ds(i * _BLOCK_T, _BLOCK_T)] = jax.lax.dot_general(
        w_ref[...],
        x_ref[...],
        dimension_numbers=(((1,), (1,)), ((), ())),
        preferred_element_type=jnp.float32,
    ) + b_ref[...]


@jax.jit
def kernel(x, W, b):
    grid = (_TOKENS // _BLOCK_T,)
    out_t = pl.pallas_call(
        _router_body,
        grid=grid,
        in_specs=[
            pl.BlockSpec((_BLOCK_T, _DIM), lambda i: (i, 0)),
            pl.BlockSpec((_EXPERTS, _DIM), lambda i: (0, 0)),
            pl.BlockSpec((_EXPERTS, 1), lambda i: (0, 0)),
        ],
        out_specs=pl.BlockSpec((_EXPERTS, _TOKENS), lambda i: (0, 0)),
        out_shape=jax.ShapeDtypeStruct((_EXPERTS, _TOKENS), jnp.float32),
        compiler_params=pltpu.CompilerParams(
            dimension_semantics=("arbitrary",),
        ),
    )(x, W, b.reshape(_EXPERTS, 1))
    return out_t.T


# speedup vs baseline: 1.2034x; 1.0528x over previous
import jax
import jax.numpy as jnp
from jax.experimental import pallas as pl
from jax.experimental.pallas import tpu as pltpu

_TOKENS = 16384
_DIM = 2048
_EXPERTS = 64
_BLOCK_T = 1024


def _router_body(x_ref, w_ref, b_ref, out_ref):
    out_ref[...] = jax.lax.dot_general(
        w_ref[...],
        x_ref[...],
        dimension_numbers=(((1,), (1,)), ((), ())),
        preferred_element_type=jnp.float32,
    ) + b_ref[...].T


@jax.jit
def kernel(x, W, b):
    grid = (_TOKENS // _BLOCK_T,)
    out_t = pl.pallas_call(
        _router_body,
        grid=grid,
        in_specs=[
            pl.BlockSpec((_BLOCK_T, _DIM), lambda i: (i, 0)),
            pl.BlockSpec((_EXPERTS, _DIM), lambda i: (0, 0)),
            pl.BlockSpec((1, _EXPERTS), lambda i: (0, 0)),
        ],
        out_specs=pl.BlockSpec((_EXPERTS, _BLOCK_T), lambda i: (0, i)),
        out_shape=jax.ShapeDtypeStruct((_EXPERTS, _TOKENS), jnp.float32),
        compiler_params=pltpu.CompilerParams(
            dimension_semantics=("arbitrary",),
        ),
    )(x, W, b.reshape(1, _EXPERTS))
    return out_t.T
